# Initial kernel scaffold; baseline (speedup 1.0000x reference)
import jax
import jax.numpy as jnp
from jax.experimental import pallas as pl


def kernel(x, edge_index, t, W_gcn, b_gcn, W_t, b_t, gamma, beta):
    raise NotImplementedError("write your pallas kernel here")


# speedup vs baseline: 17.3906x; 17.3906x over previous
import functools

import jax
import jax.numpy as jnp
from jax import lax
from jax.experimental import pallas as pl
from jax.experimental.pallas import tpu as pltpu
from jax.experimental.pallas import tpu_sc as plsc

N = 10000
E = 160000
D = 256
Q = 64
NQ = 4
NR = 2
NC = 2
NS = 16
NW = NC * NS

HIST = 10240
EW = E // NW
EWP = 5008
ET = E // NS
CH = 80
K = 125
NP = 10240
RT = NP // NS
WB = 80


def _mesh():
    return plsc.VectorSubcoreMesh(
        core_axis_name="c", subcore_axis_name="s", num_cores=NC, num_subcores=NS
    )


def _deg_body(col_hbm, out_hbm, colv, hist):
    c = lax.axis_index("c")
    s = lax.axis_index("s")
    w = c * NS + s
    pltpu.sync_copy(col_hbm.at[w], colv)
    z16 = jnp.zeros((16,), jnp.float32)
    one16 = jnp.ones((16,), jnp.float32)

    def zero(i, carry):
        hist[pl.ds(i * 16, 16)] = z16
        return carry

    lax.fori_loop(0, HIST // 16, zero, 0)

    def upd(i, carry):
        idx = colv[pl.ds(i * 16, 16)]
        plsc.addupdate_scatter(hist, [idx], one16)
        return carry

    lax.fori_loop(0, EWP // 16, upd, 0)
    pltpu.sync_copy(hist, out_hbm.at[w])


def _deg_hists(col_pad):
    fn = pl.kernel(
        _deg_body,
        out_type=jax.ShapeDtypeStruct((NW, HIST), jnp.float32),
        mesh=_mesh(),
        scratch_types=[
            pltpu.VMEM((EWP,), jnp.int32),
            pltpu.VMEM((HIST,), jnp.float32),
        ],
        compiler_params=pltpu.CompilerParams(needs_layout_passes=False),
    )
    return fn(col_pad)


def _dinv_body(h_ref, o_ref):
    deg = jnp.sum(h_ref[...], axis=0, keepdims=True) + 1.0
    o_ref[...] = lax.rsqrt(deg)


def _dinv(hists):
    return pl.pallas_call(
        _dinv_body,
        out_shape=jax.ShapeDtypeStruct((1, HIST), jnp.float32),
    )(hists)


_RB = 1000
_GRID = N // _RB


def _mm_body(x_ref, w_ref, dinv_ref, t_ref, wt_ref, bt_ref, y_ref, temb_ref):
    xw = jnp.dot(x_ref[...], w_ref[...], preferred_element_type=jnp.float32)
    y = xw * dinv_ref[...]
    y_ref[0] = y[:, 0 * Q:1 * Q]
    y_ref[1] = y[:, 1 * Q:2 * Q]
    y_ref[2] = y[:, 2 * Q:3 * Q]
    y_ref[3] = y[:, 3 * Q:4 * Q]

    @pl.when(pl.program_id(0) == 0)
    def _():
        te = jnp.dot(t_ref[...], wt_ref[...], preferred_element_type=jnp.float32)
        temb_ref[...] = jnp.maximum(te + bt_ref[...], 0.0)


def _mm(x, W_gcn, dinv_col, t, W_t, bt2):
    return pl.pallas_call(
        _mm_body,
        grid=(_GRID,),
        in_specs=[
            pl.BlockSpec((_RB, D), lambda j: (j, 0)),
            pl.BlockSpec((D, D), lambda j: (0, 0)),
            pl.BlockSpec((_RB, 1), lambda j: (j, 0)),
            pl.BlockSpec((1, D), lambda j: (0, 0)),
            pl.BlockSpec((D, D), lambda j: (0, 0)),
            pl.BlockSpec((1, D), lambda j: (0, 0)),
        ],
        out_specs=[
            pl.BlockSpec((NQ, _RB, Q), lambda j: (0, j, 0)),
            pl.BlockSpec((1, D), lambda j: (0, 0)),
        ],
        out_shape=[
            jax.ShapeDtypeStruct((NQ, N, Q), jnp.float32),
            jax.ShapeDtypeStruct((1, D), jnp.float32),
        ],
    )(x, W_gcn, dinv_col, t, W_t, bt2)


def _scat_body(y_hbm, row_hbm, col_hbm, out_hbm,
               rowi, coli, buf, zbuf, wbuf, acc_sh, sem0, sem1):
    c = lax.axis_index("c")
    s = lax.axis_index("s")
    pltpu.sync_copy(row_hbm.at[s], rowi)
    pltpu.sync_copy(col_hbm.at[s], coli)

    z16 = jnp.zeros((16,), jnp.float32)

    def zrow(i, carry):
        def zcol(k, carry2):
            zbuf[i, pl.ds(k * 16, 16)] = z16
            return carry2
        return lax.fori_loop(0, Q // 16, zcol, carry)

    lax.fori_loop(0, WB, zrow, 0)

    for p in range(NR):
        q = NR * c + p
        yt = y_hbm.at[q]

        def zacc(k, carry):
            pltpu.sync_copy(zbuf, acc_sh.at[pl.ds(s * RT + k * WB, WB)])
            return carry

        lax.fori_loop(0, RT // WB, zacc, 0)
        plsc.subcore_barrier()

        pltpu.async_copy(yt.at[rowi.at[0]], buf.at[0], sem0)

        def body(i, carry):
            j0 = 2 * i
            j1 = 2 * i + 1
            pltpu.async_copy(yt.at[rowi.at[j1]], buf.at[1], sem1)
            pltpu.make_async_copy(yt.at[rowi.at[j0]], buf.at[0], sem0).wait()
            pltpu.sync_copy(buf.at[0], acc_sh.at[coli.at[j0]], add=True)

            @pl.when(i < CH // 2 - 1)
            def _():
                pltpu.async_copy(yt.at[rowi.at[j0 + 2]], buf.at[0], sem0)

            pltpu.make_async_copy(yt.at[rowi.at[j1]], buf.at[1], sem1).wait()
            pltpu.sync_copy(buf.at[1], acc_sh.at[coli.at[j1]], add=True)
            return carry

        lax.fori_loop(0, CH // 2, body, 0)
        plsc.subcore_barrier()

        def wb(k, carry):
            off = s * RT + k * WB

            @pl.when(off < N)
            def _():
                pltpu.sync_copy(acc_sh.at[pl.ds(off, WB)], wbuf)
                pltpu.sync_copy(wbuf, out_hbm.at[q].at[pl.ds(off, WB)])

            return carry

        lax.fori_loop(0, RT // WB, wb, 0)


def _scatter(y, row_b, col_b):
    fn = pl.kernel(
        _scat_body,
        out_type=jax.ShapeDtypeStruct((NQ, N, Q), jnp.float32),
        mesh=_mesh(),
        scratch_types=[
            pltpu.VMEM((CH, K), jnp.int32),
            pltpu.VMEM((CH, K), jnp.int32),
            pltpu.VMEM((2, K, Q), jnp.float32),
            pltpu.VMEM((WB, Q), jnp.float32),
            pltpu.VMEM((WB, Q), jnp.float32),
            pltpu.VMEM_SHARED((NP, Q), jnp.float32),
            pltpu.SemaphoreType.DMA,
            pltpu.SemaphoreType.DMA,
        ],
        compiler_params=pltpu.CompilerParams(
            needs_layout_passes=False, use_tc_tiling_on_sc=False
        ),
    )
    return fn(y, row_b, col_b)


def _fin1_body(acc_ref, y_ref, dinv_ref, temb_ref, bg_ref,
               hr_ref, s1_ref, s2_ref, s1s, s2s):
    j = pl.program_id(0)
    a = jnp.concatenate([acc_ref[0], acc_ref[1], acc_ref[2], acc_ref[3]], axis=1)
    yy = jnp.concatenate([y_ref[0], y_ref[1], y_ref[2], y_ref[3]], axis=1)
    h = dinv_ref[...] * (a + yy) + bg_ref[...] + temb_ref[...]
    hr = jnp.maximum(h, 0.0)
    hr_ref[...] = hr

    @pl.when(j == 0)
    def _():
        s1s[...] = jnp.zeros_like(s1s)
        s2s[...] = jnp.zeros_like(s2s)

    s1s[...] += jnp.sum(hr, axis=0, keepdims=True)
    s2s[...] += jnp.sum(hr * hr, axis=0, keepdims=True)

    @pl.when(j == _GRID - 1)
    def _():
        s1_ref[...] = s1s[...]
        s2_ref[...] = s2s[...]


def _fin1(acc, y, dinv_col, temb, bg2):
    return pl.pallas_call(
        _fin1_body,
        grid=(_GRID,),
        in_specs=[
            pl.BlockSpec((NQ, _RB, Q), lambda j: (0, j, 0)),
            pl.BlockSpec((NQ, _RB, Q), lambda j: (0, j, 0)),
            pl.BlockSpec((_RB, 1), lambda j: (j, 0)),
            pl.BlockSpec((1, D), lambda j: (0, 0)),
            pl.BlockSpec((1, D), lambda j: (0, 0)),
        ],
        out_specs=[
            pl.BlockSpec((_RB, D), lambda j: (j, 0)),
            pl.BlockSpec((1, D), lambda j: (0, 0)),
            pl.BlockSpec((1, D), lambda j: (0, 0)),
        ],
        out_shape=[
            jax.ShapeDtypeStruct((N, D), jnp.float32),
            jax.ShapeDtypeStruct((1, D), jnp.float32),
            jax.ShapeDtypeStruct((1, D), jnp.float32),
        ],
        scratch_shapes=[
            pltpu.VMEM((1, D), jnp.float32),
            pltpu.VMEM((1, D), jnp.float32),
        ],
    )(acc, y, dinv_col, temb, bg2)


def _fin2_body(hr_ref, s1_ref, s2_ref, g_ref, b_ref, o_ref):
    mean = s1_ref[...] * (1.0 / N)
    var = s2_ref[...] * (1.0 / N) - mean * mean
    a = g_ref[...] * lax.rsqrt(var + 1e-5)
    o_ref[...] = (hr_ref[...] - mean) * a + b_ref[...]


def _fin2(hr, s1, s2, g2, b2):
    return pl.pallas_call(
        _fin2_body,
        grid=(_GRID,),
        in_specs=[
            pl.BlockSpec((_RB, D), lambda j: (j, 0)),
            pl.BlockSpec((1, D), lambda j: (0, 0)),
            pl.BlockSpec((1, D), lambda j: (0, 0)),
            pl.BlockSpec((1, D), lambda j: (0, 0)),
            pl.BlockSpec((1, D), lambda j: (0, 0)),
        ],
        out_specs=pl.BlockSpec((_RB, D), lambda j: (j, 0)),
        out_shape=jax.ShapeDtypeStruct((N, D), jnp.float32),
    )(hr, s1, s2, g2, b2)


def kernel(x, edge_index, t, W_gcn, b_gcn, W_t, b_t, gamma, beta):
    row = edge_index[0]
    col = edge_index[1]

    col_pad = jnp.pad(
        col.reshape(NW, EW), ((0, 0), (0, EWP - EW)), constant_values=N
    )
    row_b = row.reshape(NS, CH, K)
    col_b = col.reshape(NS, CH, K)

    hists = _deg_hists(col_pad)
    dinv_col = _dinv(hists).reshape(HIST)[:N].reshape(N, 1)

    bt2 = b_t.reshape(1, D)
    bg2 = b_gcn.reshape(1, D)
    g2 = gamma.reshape(1, D)
    be2 = beta.reshape(1, D)

    y, temb = _mm(x, W_gcn, dinv_col, t, W_t, bt2)
    acc = _scatter(y, row_b, col_b)
    hr, s1, s2 = _fin1(acc, y, dinv_col, temb, bg2)
    return _fin2(hr, s1, s2, g2, be2)
